# baseline (speedup 1.0000x reference)
import functools
import numpy as np

import jax
import jax.numpy as jnp
from jax import lax
from jax.experimental import pallas as pl
from jax.experimental.pallas import tpu as pltpu
from jax.experimental.pallas import tpu_sc as plsc

N = 10000
E = 160000
EA = 80000
A = 160000
NW = 32
NPT = 320
NPAD = NW * NPT
CE = 224
PAD = 2000
E_PAD = E + PAD
A_PAD = A + PAD
APT = A // NW
CH = 256
BN = 2048
F32 = jnp.float32
I32 = jnp.int32

_mesh = plsc.VectorSubcoreMesh(core_axis_name="c", subcore_axis_name="s")


def _wid():
    return lax.axis_index("s") * 2 + lax.axis_index("c")


_GDN = lax.GatherDimensionNumbers(offset_dims=(), collapsed_slice_dims=(0,),
                                  start_index_map=(0,))


def _bcast_lane(a, k):
    idx = jnp.full((16, 1), k, I32)
    return lax.gather(a, idx, _GDN, slice_sizes=(1,),
                      mode=lax.GatherScatterMode.PROMISE_IN_BOUNDS)


def _s0a_body(asrc, adst, eda, aa1, aa2, iv, ov, sem):
    w = _wid()
    base = w * APT
    pltpu.sync_copy(asrc.at[pl.ds(base, APT)], iv)
    pltpu.async_copy(eda.at[iv], ov, sem).wait()
    pltpu.sync_copy(ov, aa1.at[pl.ds(base, APT)])
    pltpu.sync_copy(adst.at[pl.ds(base, APT)], iv)
    pltpu.async_copy(eda.at[iv], ov, sem).wait()
    pltpu.sync_copy(ov, aa2.at[pl.ds(base, APT)])


def _s0a(angle_src, angle_dst, edge_dst_a):
    return pl.kernel(
        _s0a_body, mesh=_mesh,
        out_type=[jax.ShapeDtypeStruct((A,), I32),
                  jax.ShapeDtypeStruct((A,), I32)],
        scratch_types=[pltpu.VMEM((APT,), I32), pltpu.VMEM((APT,), I32),
                       pltpu.SemaphoreType.DMA],
    )(angle_src, angle_dst, edge_dst_a)


def _s0b_body(asrc, adst, adist, asw, d12, sw2, iv, g1, g2, g3, g4, o1, o2,
              sem):
    w = _wid()
    base = w * APT
    pltpu.sync_copy(asrc.at[pl.ds(base, APT)], iv)
    pltpu.async_copy(adist.at[iv], g1, sem).wait()
    pltpu.async_copy(asw.at[iv], g3, sem).wait()
    pltpu.sync_copy(adst.at[pl.ds(base, APT)], iv)
    pltpu.async_copy(adist.at[iv], g2, sem).wait()
    pltpu.async_copy(asw.at[iv], g4, sem).wait()
    last = APT - 16

    def body(i, c):
        off = jnp.minimum(i * 16, last)
        a = g1[pl.ds(off, 16)]
        b = g2[pl.ds(off, 16)]
        o1[pl.ds(off, 16)] = 0.5 * (a + b)
        p = g3[pl.ds(off, 16)]
        q = g4[pl.ds(off, 16)]
        o2[pl.ds(off, 16)] = 2.0 * (p * q)
        return c

    lax.fori_loop(0, (APT + 15) // 16, body, 0)
    pltpu.sync_copy(o1, d12.at[pl.ds(base, APT)])
    pltpu.sync_copy(o2, sw2.at[pl.ds(base, APT)])


def _s0b(angle_src, angle_dst, a_distances, a_switch):
    return pl.kernel(
        _s0b_body, mesh=_mesh,
        out_type=[jax.ShapeDtypeStruct((A_PAD,), F32),
                  jax.ShapeDtypeStruct((A_PAD,), F32)],
        scratch_types=[pltpu.VMEM((APT,), I32),
                       pltpu.VMEM((APT,), F32), pltpu.VMEM((APT,), F32),
                       pltpu.VMEM((APT,), F32), pltpu.VMEM((APT,), F32),
                       pltpu.VMEM((APT,), F32), pltpu.VMEM((APT,), F32),
                       pltpu.SemaphoreType.DMA],
    )(angle_src, angle_dst, a_distances, a_switch)


_NCH_S1 = A // CH
_NCW_S1 = _NCH_S1 // NW
_REM_S1 = _NCH_S1 - _NCW_S1 * NW


def _s1_body(aa1, aa2, edst, afvt, out, eaf, i1, i2, r1, r2, ov, cvt_f, sem):
    w = _wid()
    nch = _NCW_S1 + (w < _REM_S1).astype(I32)

    def chunk(c, carry):
        base = (c * NW + w) * CH
        pltpu.sync_copy(aa1.at[pl.ds(base, CH)], i1)
        pltpu.sync_copy(aa2.at[pl.ds(base, CH)], i2)
        cp1 = pltpu.async_copy(afvt.at[i1], r1, sem)
        cp2 = pltpu.async_copy(afvt.at[i2], r2, sem)
        cp1.wait()
        cp2.wait()

        def row(i, cc):
            for j in range(2):
                ii = i * 2 + j
                a = r1[ii, pl.ds(0, 16)]
                b = r2[ii, pl.ds(0, 16)]
                ov[ii, pl.ds(0, 16)] = a * b
                ov[ii, pl.ds(16, 16)] = a + b
            return cc

        lax.fori_loop(0, CH // 2, row, 0)
        pltpu.sync_copy(ov, out.at[pl.ds(base, CH)])

        pltpu.sync_copy(edst.at[pl.ds(base, CH)], i1)
        pltpu.async_copy(afvt.at[i1], r1, sem).wait()

        def row2(i, cc):
            for j in range(4):
                ii = i * 4 + j
                cvt_f[pl.ds(ii * 16, 16)] = r1[ii, pl.ds(0, 16)]
            return cc

        lax.fori_loop(0, CH // 4, row2, 0)
        pltpu.sync_copy(cvt_f, eaf.at[pl.ds(base * 16, CH * 16)])
        return carry

    lax.fori_loop(0, nch, chunk, 0)


def _s1(aa1, aa2, edst_p, afv128):
    return pl.kernel(
        _s1_body, mesh=_mesh,
        out_type=[jax.ShapeDtypeStruct((A_PAD, 32), F32),
                  jax.ShapeDtypeStruct((E_PAD * 16,), F32)],
        scratch_types=[pltpu.VMEM((CH,), I32), pltpu.VMEM((CH,), I32),
                       pltpu.VMEM((CH, 128), F32), pltpu.VMEM((CH, 128), F32),
                       pltpu.VMEM((CH, 32), F32),
                       pltpu.VMEM((CH * 16,), F32),
                       pltpu.SemaphoreType.DMA],
    )(aa1, aa2, edst_p, afv128)


def _s2_body(eptr, aptr, rece, eaf, reca, avang,
             gri, gai, acc, rec_v, af_f, av_v, ptr_s, sem):
    w = _wid()
    lo = w * NPT
    lo_f = lo.astype(F32)
    zero = jnp.zeros((16,), F32)

    def phase(ptr_hbm, rec_hbm, rows_hbm, out_hbm, flat):
        def zb(i, c):
            for j in range(16):
                acc[i, pl.ds(j * 16, 16)] = zero
            return c

        lax.fori_loop(0, NPT, zb, 0)
        pltpu.sync_copy(ptr_hbm, ptr_s)
        pv = ptr_s[pl.ds(w, 16)]
        p0 = pv[0]
        p1 = pv[1]
        p0a = (p0 // 8) * 8
        nch = (p1 - p0a + CE - 1) // CE

        def chunk(c, carry):
            base = p0a + c * CE
            pltpu.sync_copy(rec_hbm.at[pl.ds(base * 32, CE * 32)], rec_v)
            if flat:
                pltpu.sync_copy(rows_hbm.at[pl.ds(base * 16, CE * 16)], af_f)
            else:
                pltpu.sync_copy(rows_hbm.at[pl.ds(base, CE)], av_v)

            def edge(i, cc):
                for j in range(4):
                    ii = i * 4 + j
                    e = base + ii
                    s_f = rec_v[pl.ds(ii * 32, 16)][0]
                    valid = (e >= p0) & (e < p1)
                    row = jnp.clip(s_f - lo_f, 0.0, NPT - 1.0).astype(I32)
                    scale = jnp.where(valid, 1.0, 0.0).astype(F32)
                    r = rec_v[pl.ds(ii * 32 + 2, 16)] * scale
                    if flat:
                        a = af_f[pl.ds(ii * 16, 16)]
                    else:
                        a = av_v[ii, pl.ds(0, 16)]
                    for k in range(16):
                        ak = _bcast_lane(a, k)
                        plsc.addupdate(acc.at[row, pl.ds(k * 16, 16)],
                                       ak * r)
                return cc

            lax.fori_loop(0, CE // 4, edge, 0)
            return carry

        lax.fori_loop(0, nch, chunk, 0)
        pltpu.sync_copy(acc, out_hbm.at[pl.ds(lo, NPT)])

    phase(eptr, rece, eaf, gri, True)
    phase(aptr, reca, avang, gai, False)


def _s2(eptr, aptr, rece, eaf, reca, avang):
    return pl.kernel(
        _s2_body, mesh=_mesh,
        out_type=[jax.ShapeDtypeStruct((NPAD, 256), F32),
                  jax.ShapeDtypeStruct((NPAD, 256), F32)],
        scratch_types=[pltpu.VMEM((NPT, 256), F32),
                       pltpu.VMEM((CE * 32,), F32),
                       pltpu.VMEM((CE * 16,), F32),
                       pltpu.VMEM((CE, 16), F32),
                       pltpu.VMEM((48,), I32),
                       pltpu.SemaphoreType.DMA],
    )(eptr, aptr, rece, eaf, reca, avang)


def _t0_kernel(gd_ref, gs_ref, out_ref):
    k = lax.broadcasted_iota(I32, out_ref.shape, 1).astype(F32)
    shiftR = 0.8 + (4.4 / 16.0) * k
    x2 = 16.0 * (gd_ref[...] - shiftR) ** 2
    out_ref[...] = 0.25 * jnp.exp(-x2) * gs_ref[...]


def _t0(g_distances_p, g_switch_p):
    BE = 2000
    return pl.pallas_call(
        _t0_kernel,
        grid=(E_PAD // BE,),
        in_specs=[pl.BlockSpec((BE, 1), lambda i: (i, 0)),
                  pl.BlockSpec((BE, 1), lambda i: (i, 0))],
        out_specs=pl.BlockSpec((BE, 16), lambda i: (i, 0)),
        out_shape=jax.ShapeDtypeStruct((E_PAD, 16), F32),
    )(g_distances_p, g_switch_p)


def _t1_kernel(ang_ref, d12_ref, sw2_ref, out_ref):
    col = lax.broadcasted_iota(I32, out_ref.shape, 1)
    jz = (col % 4).astype(F32)
    jd = (col // 4).astype(F32)
    shiftZ = np.float32(np.pi / 8) + jz * np.float32(np.pi / 4)
    shiftA = 0.8 + jd * np.float32(2.7 / 4)
    f1 = 0.5 + 0.5 * jnp.cos(ang_ref[...] - shiftZ)
    f1 = f1 * f1
    f1 = f1 * f1
    f1 = f1 * f1
    f1 = f1 * f1
    f1 = f1 * f1
    f2 = jnp.exp(-8.0 * (d12_ref[...] - shiftA) ** 2)
    out_ref[...] = f1 * f2 * sw2_ref[...]


def _t1(a_angles_p, d12, sw2):
    BA = 2000
    return pl.pallas_call(
        _t1_kernel,
        grid=(A_PAD // BA,),
        in_specs=[pl.BlockSpec((BA, 1), lambda i: (i, 0)),
                  pl.BlockSpec((BA, 1), lambda i: (i, 0)),
                  pl.BlockSpec((BA, 1), lambda i: (i, 0))],
        out_specs=pl.BlockSpec((BA, 16), lambda i: (i, 0)),
        out_shape=jax.ShapeDtypeStruct((A_PAD, 16), F32),
    )(a_angles_p, d12.reshape(A_PAD, 1), sw2.reshape(A_PAD, 1))


def _t2_kernel(x_ref, w0, b0, w1, b1, w2, b2, out_ref):
    h = jax.nn.silu(x_ref[...] @ w0[...] + b0[...])
    h = jax.nn.silu(h @ w1[...] + b1[...])
    out_ref[...] = h @ w2[...] + b2[...]


def _t2(afv12, comb):
    BA = 2000
    (w0, b0), (w1, b1), (w2, b2) = comb
    full = lambda a: pl.BlockSpec(a.shape, lambda i: tuple(0 for _ in a.shape))
    return pl.pallas_call(
        _t2_kernel,
        grid=(A_PAD // BA,),
        in_specs=[pl.BlockSpec((BA, 32), lambda i: (i, 0)),
                  full(w0), full(b0), full(w1), full(b1), full(w2), full(b2)],
        out_specs=pl.BlockSpec((BA, 16), lambda i: (i, 0)),
        out_shape=jax.ShapeDtypeStruct((A_PAD, 16), F32),
    )(afv12, w0, b0, w1, b1, w2, b2)


def _t3_kernel(last, gri_ref, gai_ref, afv_ref, e0a, e0b, be0, e1, be1,
               e2, be2, a0, ba0, a1, ba1, a2, ba2, *rest):
    if last:
        i0, bi0, i1, bi1, i2, bi2, afv_out, mi_out = rest
    else:
        (afv_out,) = rest
    h = jax.nn.silu(gri_ref[...] @ e0a[...] + gai_ref[...] @ e0b[...]
                    + be0[...])
    h = jax.nn.silu(h @ e1[...] + be1[...])
    fi = h @ e2[...] + be2[...]
    g = jax.nn.silu(fi @ a0[...] + ba0[...])
    g = jax.nn.silu(g @ a1[...] + ba1[...])
    new_afv = afv_ref[...][:, :16] + (g @ a2[...] + ba2[...])
    pad = jnp.zeros((new_afv.shape[0], 112), F32)
    afv_out[...] = jnp.concatenate((new_afv, pad), axis=1)
    if last:
        m = jax.nn.silu(fi @ i0[...] + bi0[...])
        m = jax.nn.silu(m @ i1[...] + bi1[...])
        mi_out[...] = m @ i2[...] + bi2[...]


def _t3(gri, gai, afv128, emb, afvp, inter):
    (we0, be0), (we1, be1), (we2, be2) = emb
    (wa0, ba0), (wa1, ba1), (wa2, ba2) = afvp
    e0a, e0b = we0[:256], we0[256:]
    last = inter is not None
    full = lambda a: pl.BlockSpec(a.shape, lambda i: tuple(0 for _ in a.shape))
    args = [gri, gai, afv128, e0a, e0b, be0, we1, be1, we2, be2,
            wa0, ba0, wa1, ba1, wa2, ba2]
    in_specs = [pl.BlockSpec((BN, 256), lambda i: (i, 0)),
                pl.BlockSpec((BN, 256), lambda i: (i, 0)),
                pl.BlockSpec((BN, 128), lambda i: (i, 0))] + \
               [full(a) for a in args[3:]]
    if last:
        (wi0, bi0), (wi1, bi1), (wi2, bi2) = inter
        extra = [wi0, bi0, wi1, bi1, wi2, bi2]
        args += extra
        in_specs += [full(a) for a in extra]
        out_specs = [pl.BlockSpec((BN, 128), lambda i: (i, 0)),
                     pl.BlockSpec((BN, 128), lambda i: (i, 0))]
        out_shape = [jax.ShapeDtypeStruct((NPAD, 128), F32),
                     jax.ShapeDtypeStruct((NPAD, 128), F32)]
    else:
        out_specs = pl.BlockSpec((BN, 128), lambda i: (i, 0))
        out_shape = jax.ShapeDtypeStruct((NPAD, 128), F32)
    return pl.pallas_call(
        functools.partial(_t3_kernel, last),
        grid=(NPAD // BN,),
        in_specs=in_specs,
        out_specs=out_specs,
        out_shape=out_shape,
    )(*args)


def _t4_kernel(spc_ref, tab_ref, out_ref):
    z = lax.broadcasted_iota(I32, (spc_ref.shape[0], 128), 1)
    onehot = (spc_ref[...] == z).astype(F32)
    out_ref[...] = onehot @ tab_ref[...]


def _t4(species_pad, afv_table):
    tab_pad = jnp.zeros((128, 128), F32).at[:afv_table.shape[0], :16].set(
        afv_table)
    return pl.pallas_call(
        _t4_kernel,
        grid=(NPAD // BN,),
        in_specs=[pl.BlockSpec((BN, 1), lambda i: (i, 0)),
                  pl.BlockSpec((128, 128), lambda i: (0, 0))],
        out_specs=pl.BlockSpec((BN, 128), lambda i: (i, 0)),
        out_shape=jax.ShapeDtypeStruct((NPAD, 128), F32),
    )(species_pad, tab_pad)


def kernel(species, g_distances, g_switch, edge_src, edge_dst, a_distances, a_switch, a_angles, edge_dst_a, central_atom, angle_src, angle_dst, afv_table, params):
    zf = jnp.zeros((PAD,), F32)
    zi = jnp.zeros((PAD,), I32)
    gd_p = jnp.concatenate([g_distances, zf]).reshape(E_PAD, 1)
    gs_p = jnp.concatenate([g_switch, zf]).reshape(E_PAD, 1)
    aa_p = jnp.concatenate([a_angles, zf]).reshape(A_PAD, 1)
    esrc_p = jnp.concatenate([edge_src.astype(I32), zi])
    edst_p = jnp.concatenate([edge_dst.astype(I32), zi])
    cat_p = jnp.concatenate([central_atom.astype(I32), zi])
    spc_p = jnp.concatenate([species.astype(I32),
                             jnp.zeros((NPAD - N,), I32)]).reshape(NPAD, 1)

    bounds = (jnp.arange(33, dtype=I32) * NPT).clip(max=N)
    eptr = jnp.concatenate([
        jnp.searchsorted(edge_src, bounds).astype(I32),
        jnp.full((15,), E, I32)])
    aptr = jnp.concatenate([
        jnp.searchsorted(central_atom, bounds).astype(I32),
        jnp.full((15,), A, I32)])

    rad = _t0(gd_p, gs_p)
    aa1, aa2 = _s0a(angle_src.astype(I32), angle_dst.astype(I32),
                    edge_dst_a.astype(I32))
    d12, sw2 = _s0b(angle_src.astype(I32), angle_dst.astype(I32),
                    a_distances, a_switch)
    ang = _t1(aa_p, d12, sw2)
    afv128 = _t4(spc_p, afv_table)

    rece = jnp.zeros((E_PAD, 32), F32)
    rece = rece.at[:, 0].set(esrc_p.astype(F32))
    rece = rece.at[:, 2:18].set(rad)
    rece = rece.reshape(-1)
    reca = jnp.zeros((A_PAD, 32), F32)
    reca = reca.at[:, 0].set(cat_p.astype(F32))
    reca = reca.at[:, 2:18].set(ang)
    reca = reca.reshape(-1)

    mi = None
    for layer in range(3):
        afv12, eaf = _s1(aa1, aa2, edst_p, afv128)
        avang = _t2(afv12, params['comb'][layer])
        gri, gai = _s2(eptr, aptr, rece, eaf, reca, avang)
        if layer < 2:
            afv128 = _t3(gri, gai, afv128, params['emb'][layer],
                         params['afv'][layer], None)
        else:
            afv128, mi = _t3(gri, gai, afv128, params['emb'][layer],
                             params['afv'][layer], params['inter'])
    return (mi[:N], afv128[:N, :16])


# speedup vs baseline: 1.1285x; 1.1285x over previous
import functools
import numpy as np

import jax
import jax.numpy as jnp
from jax import lax
from jax.experimental import pallas as pl
from jax.experimental.pallas import tpu as pltpu
from jax.experimental.pallas import tpu_sc as plsc

N = 10000
E = 160000
EA = 80000
A = 160000
NW = 32
NPT = 320
NPAD = NW * NPT
CE = 224
PAD = 2000
E_PAD = E + PAD
A_PAD = A + PAD
APT = A // NW
CH = 256
BN = 2048
F32 = jnp.float32
I32 = jnp.int32

_mesh = plsc.VectorSubcoreMesh(core_axis_name="c", subcore_axis_name="s")


def _wid():
    return lax.axis_index("s") * 2 + lax.axis_index("c")


_GDN = lax.GatherDimensionNumbers(offset_dims=(), collapsed_slice_dims=(0,),
                                  start_index_map=(0,))


def _bcast_lane(a, k):
    idx = jnp.full((16, 1), k, I32)
    return lax.gather(a, idx, _GDN, slice_sizes=(1,),
                      mode=lax.GatherScatterMode.PROMISE_IN_BOUNDS)


def _s0a_body(asrc, adst, eda, aa1, aa2, iv, ov, sem):
    w = _wid()
    base = w * APT
    pltpu.sync_copy(asrc.at[pl.ds(base, APT)], iv)
    pltpu.async_copy(eda.at[iv], ov, sem).wait()
    pltpu.sync_copy(ov, aa1.at[pl.ds(base, APT)])
    pltpu.sync_copy(adst.at[pl.ds(base, APT)], iv)
    pltpu.async_copy(eda.at[iv], ov, sem).wait()
    pltpu.sync_copy(ov, aa2.at[pl.ds(base, APT)])


def _s0a(angle_src, angle_dst, edge_dst_a):
    return pl.kernel(
        _s0a_body, mesh=_mesh,
        out_type=[jax.ShapeDtypeStruct((A,), I32),
                  jax.ShapeDtypeStruct((A,), I32)],
        scratch_types=[pltpu.VMEM((APT,), I32), pltpu.VMEM((APT,), I32),
                       pltpu.SemaphoreType.DMA],
    )(angle_src, angle_dst, edge_dst_a)


def _s0b_body(asrc, adst, adist, asw, d12, sw2, iv, g1, g2, g3, g4, o1, o2,
              sem):
    w = _wid()
    base = w * APT
    pltpu.sync_copy(asrc.at[pl.ds(base, APT)], iv)
    pltpu.async_copy(adist.at[iv], g1, sem).wait()
    pltpu.async_copy(asw.at[iv], g3, sem).wait()
    pltpu.sync_copy(adst.at[pl.ds(base, APT)], iv)
    pltpu.async_copy(adist.at[iv], g2, sem).wait()
    pltpu.async_copy(asw.at[iv], g4, sem).wait()
    last = APT - 16

    def body(i, c):
        off = jnp.minimum(i * 16, last)
        a = g1[pl.ds(off, 16)]
        b = g2[pl.ds(off, 16)]
        o1[pl.ds(off, 16)] = 0.5 * (a + b)
        p = g3[pl.ds(off, 16)]
        q = g4[pl.ds(off, 16)]
        o2[pl.ds(off, 16)] = 2.0 * (p * q)
        return c

    lax.fori_loop(0, (APT + 15) // 16, body, 0)
    pltpu.sync_copy(o1, d12.at[pl.ds(base, APT)])
    pltpu.sync_copy(o2, sw2.at[pl.ds(base, APT)])


def _s0b(angle_src, angle_dst, a_distances, a_switch):
    return pl.kernel(
        _s0b_body, mesh=_mesh,
        out_type=[jax.ShapeDtypeStruct((A_PAD,), F32),
                  jax.ShapeDtypeStruct((A_PAD,), F32)],
        scratch_types=[pltpu.VMEM((APT,), I32),
                       pltpu.VMEM((APT,), F32), pltpu.VMEM((APT,), F32),
                       pltpu.VMEM((APT,), F32), pltpu.VMEM((APT,), F32),
                       pltpu.VMEM((APT,), F32), pltpu.VMEM((APT,), F32),
                       pltpu.SemaphoreType.DMA],
    )(angle_src, angle_dst, a_distances, a_switch)


_NCH_S1 = A // CH
_NCW_S1 = _NCH_S1 // NW
_REM_S1 = _NCH_S1 - _NCW_S1 * NW


def _s1_body(aa1, aa2, edst, afvt, out, eaf, i1, i2, r1, r2, ov, cvt_f, sem):
    w = _wid()
    nch = _NCW_S1 + (w < _REM_S1).astype(I32)

    def chunk(c, carry):
        base = (c * NW + w) * CH
        pltpu.sync_copy(aa1.at[pl.ds(base, CH)], i1)
        pltpu.sync_copy(aa2.at[pl.ds(base, CH)], i2)
        cp1 = pltpu.async_copy(afvt.at[i1], r1, sem)
        cp2 = pltpu.async_copy(afvt.at[i2], r2, sem)
        cp1.wait()
        cp2.wait()

        def row(i, cc):
            for j in range(2):
                ii = i * 2 + j
                a = r1[ii, pl.ds(0, 16)]
                b = r2[ii, pl.ds(0, 16)]
                ov[ii, pl.ds(0, 16)] = a * b
                ov[ii, pl.ds(16, 16)] = a + b
            return cc

        lax.fori_loop(0, CH // 2, row, 0)
        pltpu.sync_copy(ov, out.at[pl.ds(base, CH)])

        pltpu.sync_copy(edst.at[pl.ds(base, CH)], i1)
        pltpu.async_copy(afvt.at[i1], r1, sem).wait()

        def row2(i, cc):
            for j in range(4):
                ii = i * 4 + j
                cvt_f[pl.ds(ii * 16, 16)] = r1[ii, pl.ds(0, 16)]
            return cc

        lax.fori_loop(0, CH // 4, row2, 0)
        pltpu.sync_copy(cvt_f, eaf.at[pl.ds(base * 16, CH * 16)])
        return carry

    lax.fori_loop(0, nch, chunk, 0)


def _s1(aa1, aa2, edst_p, afv128):
    return pl.kernel(
        _s1_body, mesh=_mesh,
        out_type=[jax.ShapeDtypeStruct((A_PAD, 32), F32),
                  jax.ShapeDtypeStruct((E_PAD * 16,), F32)],
        scratch_types=[pltpu.VMEM((CH,), I32), pltpu.VMEM((CH,), I32),
                       pltpu.VMEM((CH, 128), F32), pltpu.VMEM((CH, 128), F32),
                       pltpu.VMEM((CH, 32), F32),
                       pltpu.VMEM((CH * 16,), F32),
                       pltpu.SemaphoreType.DMA],
    )(aa1, aa2, edst_p, afv128)


def _s2_body(flat, ptr_hbm, rec_hbm, rows_hbm, out_hbm,
             acc, rec_v, af_f, av_v, ptr_s, sem):
    w = _wid()
    lo = w * NPT
    lo_f = lo.astype(F32)
    zero = jnp.zeros((16,), F32)

    if True:
        def zb(i, c):
            for j in range(16):
                acc[i, pl.ds(j * 16, 16)] = zero
            return c

        lax.fori_loop(0, NPT, zb, 0)
        pltpu.sync_copy(ptr_hbm, ptr_s)
        pv = ptr_s[pl.ds(w, 16)]
        p0 = pv[0]
        p1 = pv[1]
        p0a = (p0 // 8) * 8
        nch = (p1 - p0a + CE - 1) // CE

        def chunk(c, carry):
            base = p0a + c * CE
            pltpu.sync_copy(rec_hbm.at[pl.ds(base * 32, CE * 32)], rec_v)
            if flat:
                pltpu.sync_copy(rows_hbm.at[pl.ds(base * 16, CE * 16)], af_f)
            else:
                pltpu.sync_copy(rows_hbm.at[pl.ds(base, CE)], av_v)

            def edge(i, cc):
                for j in range(4):
                    ii = i * 4 + j
                    e = base + ii
                    s_f = rec_v[pl.ds(ii * 32, 16)][0]
                    valid = (e >= p0) & (e < p1)
                    row = jnp.clip(s_f - lo_f, 0.0, NPT - 1.0).astype(I32)
                    scale = jnp.where(valid, 1.0, 0.0).astype(F32)
                    r = rec_v[pl.ds(ii * 32 + 2, 16)] * scale
                    if flat:
                        a = af_f[pl.ds(ii * 16, 16)]
                    else:
                        a = av_v[ii, pl.ds(0, 16)]
                    for k in range(16):
                        ak = _bcast_lane(a, k)
                        plsc.addupdate(acc.at[row, pl.ds(k * 16, 16)],
                                       ak * r)
                return cc

            lax.fori_loop(0, CE // 4, edge, 0)
            return carry

        lax.fori_loop(0, nch, chunk, 0)
        pltpu.sync_copy(acc, out_hbm.at[pl.ds(lo, NPT)])


def _s2_one(flat, ptr, rec, rows):
    return pl.kernel(
        functools.partial(_s2_body, flat), mesh=_mesh,
        out_type=jax.ShapeDtypeStruct((NPAD, 256), F32),
        scratch_types=[pltpu.VMEM((NPT, 256), F32),
                       pltpu.VMEM((CE * 32,), F32),
                       pltpu.VMEM((CE * 16,), F32),
                       pltpu.VMEM((CE, 16), F32),
                       pltpu.VMEM((48,), I32),
                       pltpu.SemaphoreType.DMA],
    )(ptr, rec, rows)


def _t0_kernel(gd_ref, gs_ref, out_ref):
    k = lax.broadcasted_iota(I32, out_ref.shape, 1).astype(F32)
    shiftR = 0.8 + (4.4 / 16.0) * k
    x2 = 16.0 * (gd_ref[...] - shiftR) ** 2
    out_ref[...] = 0.25 * jnp.exp(-x2) * gs_ref[...]


def _t0(g_distances_p, g_switch_p):
    BE = 2000
    return pl.pallas_call(
        _t0_kernel,
        grid=(E_PAD // BE,),
        in_specs=[pl.BlockSpec((BE, 1), lambda i: (i, 0)),
                  pl.BlockSpec((BE, 1), lambda i: (i, 0))],
        out_specs=pl.BlockSpec((BE, 16), lambda i: (i, 0)),
        out_shape=jax.ShapeDtypeStruct((E_PAD, 16), F32),
    )(g_distances_p, g_switch_p)


def _t1_kernel(ang_ref, d12_ref, sw2_ref, out_ref):
    col = lax.broadcasted_iota(I32, out_ref.shape, 1)
    jz = (col % 4).astype(F32)
    jd = (col // 4).astype(F32)
    shiftZ = np.float32(np.pi / 8) + jz * np.float32(np.pi / 4)
    shiftA = 0.8 + jd * np.float32(2.7 / 4)
    f1 = 0.5 + 0.5 * jnp.cos(ang_ref[...] - shiftZ)
    f1 = f1 * f1
    f1 = f1 * f1
    f1 = f1 * f1
    f1 = f1 * f1
    f1 = f1 * f1
    f2 = jnp.exp(-8.0 * (d12_ref[...] - shiftA) ** 2)
    out_ref[...] = f1 * f2 * sw2_ref[...]


def _t1(a_angles_p, d12, sw2):
    BA = 2000
    return pl.pallas_call(
        _t1_kernel,
        grid=(A_PAD // BA,),
        in_specs=[pl.BlockSpec((BA, 1), lambda i: (i, 0)),
                  pl.BlockSpec((BA, 1), lambda i: (i, 0)),
                  pl.BlockSpec((BA, 1), lambda i: (i, 0))],
        out_specs=pl.BlockSpec((BA, 16), lambda i: (i, 0)),
        out_shape=jax.ShapeDtypeStruct((A_PAD, 16), F32),
    )(a_angles_p, d12.reshape(A_PAD, 1), sw2.reshape(A_PAD, 1))


def _t2_kernel(x_ref, w0, b0, w1, b1, w2, b2, out_ref):
    h = jax.nn.silu(x_ref[...] @ w0[...] + b0[...])
    h = jax.nn.silu(h @ w1[...] + b1[...])
    out_ref[...] = h @ w2[...] + b2[...]


def _t2(afv12, comb):
    BA = 2000
    (w0, b0), (w1, b1), (w2, b2) = comb
    full = lambda a: pl.BlockSpec(a.shape, lambda i: tuple(0 for _ in a.shape))
    return pl.pallas_call(
        _t2_kernel,
        grid=(A_PAD // BA,),
        in_specs=[pl.BlockSpec((BA, 32), lambda i: (i, 0)),
                  full(w0), full(b0), full(w1), full(b1), full(w2), full(b2)],
        out_specs=pl.BlockSpec((BA, 16), lambda i: (i, 0)),
        out_shape=jax.ShapeDtypeStruct((A_PAD, 16), F32),
    )(afv12, w0, b0, w1, b1, w2, b2)


def _t3_kernel(last, gri_ref, gai_ref, afv_ref, e0a, e0b, be0, e1, be1,
               e2, be2, a0, ba0, a1, ba1, a2, ba2, *rest):
    if last:
        i0, bi0, i1, bi1, i2, bi2, afv_out, mi_out = rest
    else:
        (afv_out,) = rest
    h = jax.nn.silu(gri_ref[...] @ e0a[...] + gai_ref[...] @ e0b[...]
                    + be0[...])
    h = jax.nn.silu(h @ e1[...] + be1[...])
    fi = h @ e2[...] + be2[...]
    g = jax.nn.silu(fi @ a0[...] + ba0[...])
    g = jax.nn.silu(g @ a1[...] + ba1[...])
    new_afv = afv_ref[...][:, :16] + (g @ a2[...] + ba2[...])
    pad = jnp.zeros((new_afv.shape[0], 112), F32)
    afv_out[...] = jnp.concatenate((new_afv, pad), axis=1)
    if last:
        m = jax.nn.silu(fi @ i0[...] + bi0[...])
        m = jax.nn.silu(m @ i1[...] + bi1[...])
        mi_out[...] = m @ i2[...] + bi2[...]


def _t3(gri, gai, afv128, emb, afvp, inter):
    (we0, be0), (we1, be1), (we2, be2) = emb
    (wa0, ba0), (wa1, ba1), (wa2, ba2) = afvp
    e0a, e0b = we0[:256], we0[256:]
    last = inter is not None
    full = lambda a: pl.BlockSpec(a.shape, lambda i: tuple(0 for _ in a.shape))
    args = [gri, gai, afv128, e0a, e0b, be0, we1, be1, we2, be2,
            wa0, ba0, wa1, ba1, wa2, ba2]
    in_specs = [pl.BlockSpec((BN, 256), lambda i: (i, 0)),
                pl.BlockSpec((BN, 256), lambda i: (i, 0)),
                pl.BlockSpec((BN, 128), lambda i: (i, 0))] + \
               [full(a) for a in args[3:]]
    if last:
        (wi0, bi0), (wi1, bi1), (wi2, bi2) = inter
        extra = [wi0, bi0, wi1, bi1, wi2, bi2]
        args += extra
        in_specs += [full(a) for a in extra]
        out_specs = [pl.BlockSpec((BN, 128), lambda i: (i, 0)),
                     pl.BlockSpec((BN, 128), lambda i: (i, 0))]
        out_shape = [jax.ShapeDtypeStruct((NPAD, 128), F32),
                     jax.ShapeDtypeStruct((NPAD, 128), F32)]
    else:
        out_specs = pl.BlockSpec((BN, 128), lambda i: (i, 0))
        out_shape = jax.ShapeDtypeStruct((NPAD, 128), F32)
    return pl.pallas_call(
        functools.partial(_t3_kernel, last),
        grid=(NPAD // BN,),
        in_specs=in_specs,
        out_specs=out_specs,
        out_shape=out_shape,
    )(*args)


def _t4_kernel(spc_ref, tab_ref, out_ref):
    z = lax.broadcasted_iota(I32, (spc_ref.shape[0], 128), 1)
    onehot = (spc_ref[...] == z).astype(F32)
    out_ref[...] = onehot @ tab_ref[...]


def _t4(species_pad, afv_table):
    tab_pad = jnp.zeros((128, 128), F32).at[:afv_table.shape[0], :16].set(
        afv_table)
    return pl.pallas_call(
        _t4_kernel,
        grid=(NPAD // BN,),
        in_specs=[pl.BlockSpec((BN, 1), lambda i: (i, 0)),
                  pl.BlockSpec((128, 128), lambda i: (0, 0))],
        out_specs=pl.BlockSpec((BN, 128), lambda i: (i, 0)),
        out_shape=jax.ShapeDtypeStruct((NPAD, 128), F32),
    )(species_pad, tab_pad)


def kernel(species, g_distances, g_switch, edge_src, edge_dst, a_distances, a_switch, a_angles, edge_dst_a, central_atom, angle_src, angle_dst, afv_table, params):
    zf = jnp.zeros((PAD,), F32)
    zi = jnp.zeros((PAD,), I32)
    gd_p = jnp.concatenate([g_distances, zf]).reshape(E_PAD, 1)
    gs_p = jnp.concatenate([g_switch, zf]).reshape(E_PAD, 1)
    aa_p = jnp.concatenate([a_angles, zf]).reshape(A_PAD, 1)
    esrc_p = jnp.concatenate([edge_src.astype(I32), zi])
    edst_p = jnp.concatenate([edge_dst.astype(I32), zi])
    cat_p = jnp.concatenate([central_atom.astype(I32), zi])
    spc_p = jnp.concatenate([species.astype(I32),
                             jnp.zeros((NPAD - N,), I32)]).reshape(NPAD, 1)

    bounds = (jnp.arange(33, dtype=I32) * NPT).clip(max=N)
    eptr = jnp.concatenate([
        jnp.searchsorted(edge_src, bounds).astype(I32),
        jnp.full((15,), E, I32)])
    aptr = jnp.concatenate([
        jnp.searchsorted(central_atom, bounds).astype(I32),
        jnp.full((15,), A, I32)])

    rad = _t0(gd_p, gs_p)
    aa1, aa2 = _s0a(angle_src.astype(I32), angle_dst.astype(I32),
                    edge_dst_a.astype(I32))
    d12, sw2 = _s0b(angle_src.astype(I32), angle_dst.astype(I32),
                    a_distances, a_switch)
    ang = _t1(aa_p, d12, sw2)
    afv128 = _t4(spc_p, afv_table)

    rece = jnp.zeros((E_PAD, 32), F32)
    rece = rece.at[:, 0].set(esrc_p.astype(F32))
    rece = rece.at[:, 2:18].set(rad)
    rece = rece.reshape(-1)
    reca = jnp.zeros((A_PAD, 32), F32)
    reca = reca.at[:, 0].set(cat_p.astype(F32))
    reca = reca.at[:, 2:18].set(ang)
    reca = reca.reshape(-1)

    mi = None
    for layer in range(3):
        afv12, eaf = _s1(aa1, aa2, edst_p, afv128)
        avang = _t2(afv12, params['comb'][layer])
        gri = _s2_one(True, eptr, rece, eaf)
        gai = _s2_one(False, aptr, reca, avang)
        if layer < 2:
            afv128 = _t3(gri, gai, afv128, params['emb'][layer],
                         params['afv'][layer], None)
        else:
            afv128, mi = _t3(gri, gai, afv128, params['emb'][layer],
                             params['afv'][layer], params['inter'])
    return (mi[:N], afv128[:N, :16])
